# Initial kernel scaffold; baseline (speedup 1.0000x reference)
import jax
import jax.numpy as jnp
from jax.experimental import pallas as pl


def kernel(inputs, A_real, A_imag):
    raise NotImplementedError("write your pallas kernel here")


# speedup vs baseline: 31.1559x; 31.1559x over previous
import functools

import jax
import jax.numpy as jnp
from jax import lax
from jax.experimental import pallas as pl
from jax.experimental.pallas import tpu as pltpu
from jax.experimental.pallas import tpu_sc as plsc

_NC = 2
_NS = 16
_L = 16


def _recurrence_kernel(B, T, C, CT):
    NW = _NC * _NS
    CH = (B * C) // NW
    KV = CH // _L
    SUB_PER_B = C // CH

    mesh = plsc.VectorSubcoreMesh(core_axis_name="c", subcore_axis_name="s")

    @functools.partial(
        pl.kernel,
        mesh=mesh,
        compiler_params=pltpu.CompilerParams(needs_layout_passes=False),
        out_type=jax.ShapeDtypeStruct((B, T, 2 * C), jnp.float32),
        scratch_types=[
            pltpu.VMEM((CT, CH), jnp.float32),
            pltpu.VMEM((CT, 2 * CH), jnp.float32),
            pltpu.VMEM((CH,), jnp.float32),
            pltpu.VMEM((CH,), jnp.float32),
        ],
    )
    def kern(x_hbm, ar_hbm, ai_hbm, out_hbm, xbuf, obuf, arbuf, aibuf):
        wid = lax.axis_index("s") * _NC + lax.axis_index("c")
        b = wid // SUB_PER_B
        c0 = (wid % SUB_PER_B) * CH

        pltpu.sync_copy(ar_hbm.at[pl.ds(c0, CH)], arbuf)
        pltpu.sync_copy(ai_hbm.at[pl.ds(c0, CH)], aibuf)

        ar = [arbuf[pl.ds(k * _L, _L)] for k in range(KV)]
        ai = [aibuf[pl.ds(k * _L, _L)] for k in range(KV)]

        iot = lax.iota(jnp.int32, _L)
        col_r = [2 * (k * _L) + 2 * iot for k in range(KV)]
        col_i = [2 * (k * _L) + 2 * iot + 1 for k in range(KV)]

        sr = [jnp.zeros((_L,), jnp.float32) for _ in range(KV)]
        si = [jnp.zeros((_L,), jnp.float32) for _ in range(KV)]

        def step(t, carry):
            sr, si = carry
            tvec = jnp.full((_L,), t, jnp.int32)
            nsr, nsi = [], []
            for k in range(KV):
                x = xbuf[t, pl.ds(k * _L, _L)]
                nr = sr[k] * ar[k] - si[k] * ai[k] + x
                ni = ai[k] * sr[k] + ar[k] * si[k]
                plsc.store_scatter(obuf, [tvec, col_r[k]], nr)
                plsc.store_scatter(obuf, [tvec, col_i[k]], ni)
                nsr.append(nr)
                nsi.append(ni)
            return tuple(nsr), tuple(nsi)

        carry = (tuple(sr), tuple(si))
        for chunk in range(T // CT):
            t0 = chunk * CT
            pltpu.sync_copy(x_hbm.at[b, pl.ds(t0, CT), pl.ds(c0, CH)], xbuf)
            carry = lax.fori_loop(0, CT, step, carry)
            pltpu.sync_copy(obuf, out_hbm.at[b, pl.ds(t0, CT), pl.ds(2 * c0, 2 * CH)])

    return kern


def kernel(inputs, A_real, A_imag):
    B, T, C = inputs.shape
    CT = 256
    out = _recurrence_kernel(B, T, C, CT)(inputs, A_real, A_imag)
    return out.reshape(B, T, C, 2)


# speedup vs baseline: 34.6225x; 1.1113x over previous
import functools

import jax
import jax.numpy as jnp
from jax import lax
from jax.experimental import pallas as pl
from jax.experimental.pallas import tpu as pltpu
from jax.experimental.pallas import tpu_sc as plsc

_NC = 2
_NS = 16
_L = 16


def _recurrence_kernel(B, T, C, CT):
    NW = _NC * _NS
    CH = (B * C) // NW
    KV = CH // _L
    SUB_PER_B = C // CH
    NCHUNK = T // CT

    mesh = plsc.VectorSubcoreMesh(core_axis_name="c", subcore_axis_name="s")

    @functools.partial(
        pl.kernel,
        mesh=mesh,
        compiler_params=pltpu.CompilerParams(needs_layout_passes=False),
        out_type=jax.ShapeDtypeStruct((B, T, 2 * C), jnp.float32),
        scratch_types=[
            pltpu.VMEM((CT, CH), jnp.float32),
            pltpu.VMEM((CT, CH), jnp.float32),
            pltpu.VMEM((CT, 2 * CH), jnp.float32),
            pltpu.VMEM((CT, 2 * CH), jnp.float32),
            pltpu.VMEM((CH,), jnp.float32),
            pltpu.VMEM((CH,), jnp.float32),
            pltpu.SemaphoreType.DMA,
            pltpu.SemaphoreType.DMA,
            pltpu.SemaphoreType.DMA,
            pltpu.SemaphoreType.DMA,
        ],
    )
    def kern(x_hbm, ar_hbm, ai_hbm, out_hbm, xbuf0, xbuf1, obuf0, obuf1,
             arbuf, aibuf, sin0, sin1, sout0, sout1):
        wid = lax.axis_index("s") * _NC + lax.axis_index("c")
        b = wid // SUB_PER_B
        c0 = (wid % SUB_PER_B) * CH
        xbuf = (xbuf0, xbuf1)
        obuf = (obuf0, obuf1)
        sin = (sin0, sin1)
        sout = (sout0, sout1)

        def start_in(chunk):
            p = chunk % 2
            return pltpu.async_copy(
                x_hbm.at[b, pl.ds(chunk * CT, CT), pl.ds(c0, CH)],
                xbuf[p], sin[p])

        def start_out(chunk):
            p = chunk % 2
            return pltpu.async_copy(
                obuf[p],
                out_hbm.at[b, pl.ds(chunk * CT, CT), pl.ds(2 * c0, 2 * CH)],
                sout[p])

        in_dma = [None] * NCHUNK
        out_dma = [None] * NCHUNK
        in_dma[0] = start_in(0)
        in_dma[1] = start_in(1)

        pltpu.sync_copy(ar_hbm.at[pl.ds(c0, CH)], arbuf)
        pltpu.sync_copy(ai_hbm.at[pl.ds(c0, CH)], aibuf)

        ar = [arbuf[pl.ds(k * _L, _L)] for k in range(KV)]
        ai = [aibuf[pl.ds(k * _L, _L)] for k in range(KV)]

        iot = lax.iota(jnp.int32, _L)
        col_r = [2 * (k * _L) + 2 * iot for k in range(KV)]
        col_i = [2 * (k * _L) + 2 * iot + 1 for k in range(KV)]

        sr = [jnp.zeros((_L,), jnp.float32) for _ in range(KV)]
        si = [jnp.zeros((_L,), jnp.float32) for _ in range(KV)]
        state = (tuple(sr), tuple(si))
        zero16 = jnp.zeros((_L,), jnp.int32)

        def make_step(p):
            xchunk = xbuf[p]
            ochunk = obuf[p]

            def step(t, carry):
                (sr, si), tvec = carry
                nsr, nsi = [], []
                for k in range(KV):
                    x = xchunk[t, pl.ds(k * _L, _L)]
                    nr = sr[k] * ar[k] - si[k] * ai[k] + x
                    ni = ai[k] * sr[k] + ar[k] * si[k]
                    plsc.store_scatter(ochunk, [tvec, col_r[k]], nr)
                    plsc.store_scatter(ochunk, [tvec, col_i[k]], ni)
                    nsr.append(nr)
                    nsi.append(ni)
                return (tuple(nsr), tuple(nsi)), tvec + 1

            return step

        for chunk in range(NCHUNK):
            p = chunk % 2
            in_dma[chunk].wait()
            if chunk >= 2:
                out_dma[chunk - 2].wait()
            state, _ = lax.fori_loop(0, CT, make_step(p), (state, zero16))
            out_dma[chunk] = start_out(chunk)
            if chunk + 2 < NCHUNK:
                in_dma[chunk + 2] = start_in(chunk + 2)

        out_dma[NCHUNK - 2].wait()
        out_dma[NCHUNK - 1].wait()

    return kern


def kernel(inputs, A_real, A_imag):
    B, T, C = inputs.shape
    CT = 128
    out = _recurrence_kernel(B, T, C, CT)(inputs, A_real, A_imag)
    return out.reshape(B, T, C, 2)


# speedup vs baseline: 144.1236x; 4.1627x over previous
import functools

import jax
import jax.numpy as jnp
from jax import lax
from jax.experimental import pallas as pl
from jax.experimental.pallas import tpu as pltpu
from jax.experimental.pallas import tpu_sc as plsc

_NC = 2
_NS = 16
_L = 16


def _recurrence_kernel(B, T, C, CT):
    NW = _NC * _NS
    CH = (B * C) // NW
    KV = CH // _L
    SUB_PER_B = C // CH
    NCHUNK = T // CT

    mesh = plsc.VectorSubcoreMesh(core_axis_name="c", subcore_axis_name="s")

    @functools.partial(
        pl.kernel,
        mesh=mesh,
        compiler_params=pltpu.CompilerParams(needs_layout_passes=False),
        out_type=jax.ShapeDtypeStruct((B, T, 2 * SUB_PER_B, CH), jnp.float32),
        scratch_types=[
            pltpu.VMEM((CT, CH), jnp.float32),
            pltpu.VMEM((CT, CH), jnp.float32),
            pltpu.VMEM((2, CT, CH), jnp.float32),
            pltpu.VMEM((2, CT, CH), jnp.float32),
            pltpu.VMEM((CH,), jnp.float32),
            pltpu.VMEM((CH,), jnp.float32),
            pltpu.SemaphoreType.DMA,
            pltpu.SemaphoreType.DMA,
            pltpu.SemaphoreType.DMA,
            pltpu.SemaphoreType.DMA,
        ],
    )
    def kern(x_hbm, ar_hbm, ai_hbm, out_hbm, xbuf0, xbuf1, obuf0, obuf1,
             arbuf, aibuf, sin0, sin1, sout0, sout1):
        wid = lax.axis_index("s") * _NC + lax.axis_index("c")
        b = wid // SUB_PER_B
        g = wid % SUB_PER_B
        c0 = g * CH
        xbuf = (xbuf0, xbuf1)
        obuf = (obuf0, obuf1)
        sin = (sin0, sin1)
        sout = (sout0, sout1)

        def start_in(chunk):
            p = chunk % 2
            return pltpu.async_copy(
                x_hbm.at[b, pl.ds(chunk * CT, CT), pl.ds(c0, CH)],
                xbuf[p], sin[p])

        def start_out(chunk):
            p = chunk % 2
            ts = pl.ds(chunk * CT, CT)
            dr = pltpu.async_copy(
                obuf[p].at[0], out_hbm.at[b, ts, 2 * g], sout[p])
            di = pltpu.async_copy(
                obuf[p].at[1], out_hbm.at[b, ts, 2 * g + 1], sout[p])
            return (dr, di)

        in_dma = [None] * NCHUNK
        out_dma = [None] * NCHUNK
        in_dma[0] = start_in(0)
        in_dma[1] = start_in(1)

        pltpu.sync_copy(ar_hbm.at[pl.ds(c0, CH)], arbuf)
        pltpu.sync_copy(ai_hbm.at[pl.ds(c0, CH)], aibuf)

        ar = [arbuf[pl.ds(k * _L, _L)] for k in range(KV)]
        ai = [aibuf[pl.ds(k * _L, _L)] for k in range(KV)]

        sr = [jnp.zeros((_L,), jnp.float32) for _ in range(KV)]
        si = [jnp.zeros((_L,), jnp.float32) for _ in range(KV)]
        state = (tuple(sr), tuple(si))

        def make_step(p):
            xchunk = xbuf[p]
            ochunk = obuf[p]

            def step(t, carry):
                sr, si = carry
                nsr, nsi = [], []
                for k in range(KV):
                    x = xchunk[t, pl.ds(k * _L, _L)]
                    nr = sr[k] * ar[k] - si[k] * ai[k] + x
                    ni = ai[k] * sr[k] + ar[k] * si[k]
                    ochunk[0, t, pl.ds(k * _L, _L)] = nr
                    ochunk[1, t, pl.ds(k * _L, _L)] = ni
                    nsr.append(nr)
                    nsi.append(ni)
                return tuple(nsr), tuple(nsi)

            return step

        for chunk in range(NCHUNK):
            p = chunk % 2
            in_dma[chunk].wait()
            if chunk >= 2:
                d0, d1 = out_dma[chunk - 2]
                d0.wait()
                d1.wait()
            state = lax.fori_loop(0, CT, make_step(p), state)
            out_dma[chunk] = start_out(chunk)
            if chunk + 2 < NCHUNK:
                in_dma[chunk + 2] = start_in(chunk + 2)

        for chunk in (NCHUNK - 2, NCHUNK - 1):
            d0, d1 = out_dma[chunk]
            d0.wait()
            d1.wait()

    return kern


def kernel(inputs, A_real, A_imag):
    B, T, C = inputs.shape
    CT = 128
    G = C // 128
    out = _recurrence_kernel(B, T, C, CT)(inputs, A_real, A_imag)
    out = out.reshape(B, T, G, 2, 128).transpose(0, 1, 2, 4, 3)
    return out.reshape(B, T, C, 2)
